# speedup vs baseline: 18.2619x; 18.2619x over previous
import jax
import jax.numpy as jnp
from jax.experimental import pallas as pl

B = 64
P = 256
N = B * P
NVP = 8
NV = B * NVP
HID = 16
PFC_DIM = 13
K1 = 64
K2 = 16

_F32 = jnp.float32


def _mm(a, b):
    return jnp.dot(a, b, precision=jax.lax.Precision.HIGHEST,
                   preferred_element_type=_F32)


def _silu(x):
    return x * (1.0 / (1.0 + jnp.exp(-x)))


def _topk_mask(d2, k):
    d2c = jnp.maximum(d2, 0.0)
    keys = jax.lax.bitcast_convert_type(d2c, jnp.int32)
    hi0 = jnp.max(keys, axis=1, keepdims=True)
    lo0 = jnp.zeros_like(hi0)

    def body(_, lohi):
        lo, hi = lohi
        mid = lo + ((hi - lo) >> 1)
        cnt = jnp.sum((keys <= mid).astype(jnp.int32), axis=1, keepdims=True)
        pred = cnt >= k
        return (jnp.where(pred, lo, mid + 1), jnp.where(pred, mid, hi))

    _, t = jax.lax.fori_loop(0, 31, body, (lo0, hi0))
    strict = keys < t
    ties = keys == t
    need = k - jnp.sum(strict.astype(jnp.int32), axis=1, keepdims=True)
    r = jax.lax.broadcasted_iota(jnp.int32, (P, P), 0)
    c = jax.lax.broadcasted_iota(jnp.int32, (P, P), 1)
    upper = (r < c).astype(jnp.bfloat16)
    excl = jnp.dot(ties.astype(jnp.bfloat16), upper,
                   preferred_element_type=_F32)
    sel = strict | (ties & (excl < need.astype(_F32)))
    return sel.astype(_F32)


def _edge_mean(a, c, mask, k):
    s = _silu(a[:, :, None] + c.T[None, :, :])
    h = jnp.sum(s * mask[:, None, :], axis=2)
    return h * (1.0 / k)


def _pairdist(xd, xs):
    sq_d = jnp.sum(xd * xd, axis=1, keepdims=True)
    sq_s = jnp.sum(xs * xs, axis=1, keepdims=True)
    return sq_d - 2.0 * _mm(xd, xs.T) + sq_s.T


def _conv1_body(xp_ref, peW1_ref, peb1_ref, peW2_ref, peb2_ref,
                W1_ref, b1_ref, W2_ref, b2_ref, f1_ref):
    xp = xp_ref[...]
    enc = _mm(_silu(_mm(xp, peW1_ref[...]) + peb1_ref[...]),
              peW2_ref[...]) + peb2_ref[...]
    d2 = _pairdist(enc, enc)
    mask = _topk_mask(d2, K1)
    W1 = W1_ref[...]
    W1a, W1b = W1[:HID, :], W1[HID:, :]
    a = _mm(enc, W1a - W1b) + b1_ref[...]
    c = _mm(enc, W1b)
    h = _edge_mean(a, c, mask, K1)
    f1_ref[...] = _mm(h, W2_ref[...]) + b2_ref[...]


def _conv2_body(dst_ref, src_ref, W1_ref, b1_ref, W2_ref, b2_ref,
                oW1_ref, ob1_ref, oW2_ref, ob2_ref, oW3_ref, ob3_ref,
                out_ref):
    xd = dst_ref[...]
    xs = src_ref[...]
    d2 = _pairdist(xd, xs)
    mask = _topk_mask(d2, K2)
    W1 = W1_ref[...]
    dcat = xd.shape[1]
    W1a, W1b = W1[:dcat, :], W1[dcat:, :]
    a = _mm(xd, W1a - W1b) + b1_ref[...]
    c = _mm(xs, W1b)
    h = _edge_mean(a, c, mask, K2)
    f2 = _mm(h, W2_ref[...]) + b2_ref[...]
    h1 = _silu(_mm(f2, oW1_ref[...]) + ob1_ref[...])
    h2 = _silu(_mm(h1, oW2_ref[...]) + ob2_ref[...])
    out_ref[...] = _mm(h2, oW3_ref[...]) + ob3_ref[...]


def _vtx_body(xv_ref, W1_ref, b1_ref, W2_ref, b2_ref, out_ref):
    xv = xv_ref[...]
    out_ref[...] = _mm(_silu(_mm(xv, W1_ref[...]) + b1_ref[...]),
                       W2_ref[...]) + b2_ref[...]


def _full(shape):
    return pl.BlockSpec(shape, lambda b: (0, 0))


def kernel(x_pfc, x_vtx, batch_pfc, batch_vtx,
           pe_W1, pe_b1, pe_W2, pe_b2,
           ve_W1, ve_b1, ve_W2, ve_b2,
           c1_W1, c1_b1, c1_W2, c1_b2,
           c2_W1, c2_b1, c2_W2, c2_b2,
           o_W1, o_b1, o_W2, o_b2, o_W3, o_b3):
    row = lambda v: v.reshape(1, -1)

    feats1 = pl.pallas_call(
        _conv1_body,
        grid=(B,),
        in_specs=[
            pl.BlockSpec((P, PFC_DIM), lambda b: (b, 0)),
            _full(pe_W1.shape), _full((1, HID)),
            _full(pe_W2.shape), _full((1, HID)),
            _full(c1_W1.shape), _full((1, HID)),
            _full(c1_W2.shape), _full((1, HID)),
        ],
        out_specs=pl.BlockSpec((P, HID), lambda b: (b, 0)),
        out_shape=jax.ShapeDtypeStruct((N, HID), _F32),
    )(x_pfc, pe_W1, row(pe_b1), pe_W2, row(pe_b2),
      c1_W1, row(c1_b1), c1_W2, row(c1_b2))

    concat_feats = jnp.concatenate([x_pfc, feats1], axis=1)
    charged_idx = jnp.nonzero(x_pfc[:, 11] != 0, size=N, fill_value=0)[0]
    charged = concat_feats[charged_idx]

    dcat = PFC_DIM + HID
    out = pl.pallas_call(
        _conv2_body,
        grid=(B,),
        in_specs=[
            pl.BlockSpec((P, dcat), lambda b: (b, 0)),
            pl.BlockSpec((P, dcat), lambda b: (b, 0)),
            _full(c2_W1.shape), _full((1, HID)),
            _full(c2_W2.shape), _full((1, HID)),
            _full(o_W1.shape), _full((1, 32)),
            _full(o_W2.shape), _full((1, 4)),
            _full(o_W3.shape), _full((1, 1)),
        ],
        out_specs=pl.BlockSpec((P, 1), lambda b: (b, 0)),
        out_shape=jax.ShapeDtypeStruct((N, 1), _F32),
    )(concat_feats, charged, c2_W1, row(c2_b1), c2_W2, row(c2_b2),
      o_W1, row(o_b1), o_W2, row(o_b2), o_W3, row(o_b3))

    x_vtx_enc = pl.pallas_call(
        _vtx_body,
        out_shape=jax.ShapeDtypeStruct((NV, HID), _F32),
    )(x_vtx, ve_W1, row(ve_b1), ve_W2, row(ve_b2))

    return (out, batch_pfc, feats1, x_vtx_enc)


# speedup vs baseline: 18.6356x; 1.0205x over previous
import jax
import jax.numpy as jnp
from jax.experimental import pallas as pl

B = 64
P = 256
N = B * P
NVP = 8
NV = B * NVP
HID = 16
PFC_DIM = 13
K1 = 64
K2 = 16

_F32 = jnp.float32


def _mm(a, b):
    return jnp.dot(a, b, precision=jax.lax.Precision.HIGHEST,
                   preferred_element_type=_F32)


def _silu(x):
    return x * (1.0 / (1.0 + jnp.exp(-x)))


def _topk_mask(d2, k):
    d2c = jnp.maximum(d2, 0.0)
    keys = jax.lax.bitcast_convert_type(d2c, jnp.int32)
    hi0 = jnp.max(keys, axis=1, keepdims=True)
    lo0 = jnp.zeros_like(hi0)

    def body(_, lohi):
        lo, hi = lohi
        mid = lo + ((hi - lo) >> 1)
        cnt = jnp.sum((keys <= mid).astype(jnp.int32), axis=1, keepdims=True)
        pred = cnt >= k
        return (jnp.where(pred, lo, mid + 1), jnp.where(pred, mid, hi))

    _, t = jax.lax.fori_loop(0, 31, body, (lo0, hi0))
    strict = keys < t
    ties = keys == t
    need = k - jnp.sum(strict.astype(jnp.int32), axis=1, keepdims=True)
    r = jax.lax.broadcasted_iota(jnp.int32, (P, P), 0)
    c = jax.lax.broadcasted_iota(jnp.int32, (P, P), 1)
    upper = (r < c).astype(jnp.bfloat16)
    excl = jnp.dot(ties.astype(jnp.bfloat16), upper,
                   preferred_element_type=_F32)
    sel = strict | (ties & (excl < need.astype(_F32)))
    return sel.astype(_F32)


def _edge_mean(a, c, mask, k):
    s = _silu(a[:, :, None] + c.T[None, :, :])
    h = jnp.sum(s * mask[:, None, :], axis=2)
    return h * (1.0 / k)


def _pairdist(xd, xs):
    sq_d = jnp.sum(xd * xd, axis=1, keepdims=True)
    sq_s = jnp.sum(xs * xs, axis=1, keepdims=True)
    return sq_d - 2.0 * _mm(xd, xs.T) + sq_s.T


def _conv1_body(xp_ref, peW1_ref, peb1_ref, peW2_ref, peb2_ref,
                W1_ref, b1_ref, W2_ref, b2_ref, f1_ref):
    xp = xp_ref[...]
    enc = _mm(_silu(_mm(xp, peW1_ref[...]) + peb1_ref[...]),
              peW2_ref[...]) + peb2_ref[...]
    d2 = _pairdist(enc, enc)
    mask = _topk_mask(d2, K1)
    W1 = W1_ref[...]
    W1a, W1b = W1[:HID, :], W1[HID:, :]
    a = _mm(enc, W1a - W1b) + b1_ref[...]
    c = _mm(enc, W1b)
    h = _edge_mean(a, c, mask, K1)
    f1_ref[...] = _mm(h, W2_ref[...]) + b2_ref[...]


def _conv2_body(dst_ref, src_ref, W1_ref, b1_ref, W2_ref, b2_ref,
                oW1_ref, ob1_ref, oW2_ref, ob2_ref, oW3_ref, ob3_ref,
                out_ref):
    xd = dst_ref[...]
    xs = src_ref[...]
    d2 = _pairdist(xd, xs)
    mask = _topk_mask(d2, K2)
    W1 = W1_ref[...]
    dcat = xd.shape[1]
    W1a, W1b = W1[:dcat, :], W1[dcat:, :]
    a = _mm(xd, W1a - W1b) + b1_ref[...]
    c = _mm(xs, W1b)
    h = _edge_mean(a, c, mask, K2)
    f2 = _mm(h, W2_ref[...]) + b2_ref[...]
    h1 = _silu(_mm(f2, oW1_ref[...]) + ob1_ref[...])
    h2 = _silu(_mm(h1, oW2_ref[...]) + ob2_ref[...])
    out_ref[...] = _mm(h2, oW3_ref[...]) + ob3_ref[...]


def _vtx_body(xv_ref, W1_ref, b1_ref, W2_ref, b2_ref, out_ref):
    xv = xv_ref[...]
    out_ref[...] = _mm(_silu(_mm(xv, W1_ref[...]) + b1_ref[...]),
                       W2_ref[...]) + b2_ref[...]


def _full(shape):
    return pl.BlockSpec(shape, lambda b: (0, 0))


def kernel(x_pfc, x_vtx, batch_pfc, batch_vtx,
           pe_W1, pe_b1, pe_W2, pe_b2,
           ve_W1, ve_b1, ve_W2, ve_b2,
           c1_W1, c1_b1, c1_W2, c1_b2,
           c2_W1, c2_b1, c2_W2, c2_b2,
           o_W1, o_b1, o_W2, o_b2, o_W3, o_b3):
    row = lambda v: v.reshape(1, -1)

    feats1 = pl.pallas_call(
        _conv1_body,
        grid=(B,),
        in_specs=[
            pl.BlockSpec((P, PFC_DIM), lambda b: (b, 0)),
            _full(pe_W1.shape), _full((1, HID)),
            _full(pe_W2.shape), _full((1, HID)),
            _full(c1_W1.shape), _full((1, HID)),
            _full(c1_W2.shape), _full((1, HID)),
        ],
        out_specs=pl.BlockSpec((P, HID), lambda b: (b, 0)),
        out_shape=jax.ShapeDtypeStruct((N, HID), _F32),
    )(x_pfc, pe_W1, row(pe_b1), pe_W2, row(pe_b2),
      c1_W1, row(c1_b1), c1_W2, row(c1_b2))

    concat_feats = jnp.concatenate([x_pfc, feats1], axis=1)
    charged = concat_feats

    dcat = PFC_DIM + HID
    out = pl.pallas_call(
        _conv2_body,
        grid=(B,),
        in_specs=[
            pl.BlockSpec((P, dcat), lambda b: (b, 0)),
            pl.BlockSpec((P, dcat), lambda b: (b, 0)),
            _full(c2_W1.shape), _full((1, HID)),
            _full(c2_W2.shape), _full((1, HID)),
            _full(o_W1.shape), _full((1, 32)),
            _full(o_W2.shape), _full((1, 4)),
            _full(o_W3.shape), _full((1, 1)),
        ],
        out_specs=pl.BlockSpec((P, 1), lambda b: (b, 0)),
        out_shape=jax.ShapeDtypeStruct((N, 1), _F32),
    )(concat_feats, charged, c2_W1, row(c2_b1), c2_W2, row(c2_b2),
      o_W1, row(o_b1), o_W2, row(o_b2), o_W3, row(o_b3))

    x_vtx_enc = pl.pallas_call(
        _vtx_body,
        out_shape=jax.ShapeDtypeStruct((NV, HID), _F32),
    )(x_vtx, ve_W1, row(ve_b1), ve_W2, row(ve_b2))

    return (out, batch_pfc, feats1, x_vtx_enc)


# speedup vs baseline: 33.1448x; 1.7786x over previous
import jax
import jax.numpy as jnp
from jax.experimental import pallas as pl

B = 64
P = 256
N = B * P
NVP = 8
NV = B * NVP
HID = 16
PFC_DIM = 13
K1 = 64
K2 = 16

_F32 = jnp.float32


def _mm(a, b):
    return jnp.dot(a, b, precision=jax.lax.Precision.HIGHEST,
                   preferred_element_type=_F32)


def _silu(x):
    return x * (1.0 / (1.0 + jnp.exp(-x)))


def _topk_maskT(d2T, k):
    d2c = jnp.maximum(d2T, 0.0)
    keys = jax.lax.bitcast_convert_type(d2c, jnp.int32)
    hi0 = jnp.max(keys, axis=0, keepdims=True)
    lo0 = jnp.zeros_like(hi0)

    def body(_, lohi):
        lo, hi = lohi
        mid = lo + ((hi - lo) >> 1)
        cnt = jnp.sum((keys <= mid).astype(jnp.int32), axis=0, keepdims=True)
        pred = cnt >= k
        return (jnp.where(pred, lo, mid + 1), jnp.where(pred, mid, hi))

    _, t = jax.lax.fori_loop(0, 31, body, (lo0, hi0))
    strict = keys < t
    ties = keys == t
    need = k - jnp.sum(strict.astype(jnp.int32), axis=0, keepdims=True)
    r = jax.lax.broadcasted_iota(jnp.int32, (P, P), 0)
    c = jax.lax.broadcasted_iota(jnp.int32, (P, P), 1)
    lower = (c < r).astype(jnp.bfloat16)
    excl = jnp.dot(lower, ties.astype(jnp.bfloat16),
                   preferred_element_type=_F32)
    sel = strict | (ties & (excl < need.astype(_F32)))
    return sel.astype(_F32)


def _edge_meanT(a, c, maskT, k):
    s = _silu(c[:, :, None] + a.T[None, :, :])
    ht = jnp.sum(s * maskT[:, None, :], axis=0)
    return ht.T * (1.0 / k)


def _pairdistT(xd, xs):
    sq_d = jnp.sum(xd * xd, axis=1, keepdims=True)
    sq_s = jnp.sum(xs * xs, axis=1, keepdims=True)
    return (sq_s + sq_d.T) - 2.0 * _mm(xs, xd.T)


def _conv1_body(xp_ref, peW1_ref, peb1_ref, peW2_ref, peb2_ref,
                W1_ref, b1_ref, W2_ref, b2_ref, f1_ref):
    xp = xp_ref[...]
    enc = _mm(_silu(_mm(xp, peW1_ref[...]) + peb1_ref[...]),
              peW2_ref[...]) + peb2_ref[...]
    d2T = _pairdistT(enc, enc)
    maskT = _topk_maskT(d2T, K1)
    W1 = W1_ref[...]
    W1a, W1b = W1[:HID, :], W1[HID:, :]
    a = _mm(enc, W1a - W1b) + b1_ref[...]
    c = _mm(enc, W1b)
    h = _edge_meanT(a, c, maskT, K1)
    f1_ref[...] = _mm(h, W2_ref[...]) + b2_ref[...]


def _conv2_body(dst_ref, src_ref, W1_ref, b1_ref, W2_ref, b2_ref,
                oW1_ref, ob1_ref, oW2_ref, ob2_ref, oW3_ref, ob3_ref,
                out_ref):
    xd = dst_ref[...]
    xs = src_ref[...]
    d2T = _pairdistT(xd, xs)
    maskT = _topk_maskT(d2T, K2)
    W1 = W1_ref[...]
    dcat = xd.shape[1]
    W1a, W1b = W1[:dcat, :], W1[dcat:, :]
    a = _mm(xd, W1a - W1b) + b1_ref[...]
    c = _mm(xs, W1b)
    h = _edge_meanT(a, c, maskT, K2)
    f2 = _mm(h, W2_ref[...]) + b2_ref[...]
    h1 = _silu(_mm(f2, oW1_ref[...]) + ob1_ref[...])
    h2 = _silu(_mm(h1, oW2_ref[...]) + ob2_ref[...])
    out_ref[...] = _mm(h2, oW3_ref[...]) + ob3_ref[...]


def _vtx_body(xv_ref, W1_ref, b1_ref, W2_ref, b2_ref, out_ref):
    xv = xv_ref[...]
    out_ref[...] = _mm(_silu(_mm(xv, W1_ref[...]) + b1_ref[...]),
                       W2_ref[...]) + b2_ref[...]


def _full(shape):
    return pl.BlockSpec(shape, lambda b: (0, 0))


def kernel(x_pfc, x_vtx, batch_pfc, batch_vtx,
           pe_W1, pe_b1, pe_W2, pe_b2,
           ve_W1, ve_b1, ve_W2, ve_b2,
           c1_W1, c1_b1, c1_W2, c1_b2,
           c2_W1, c2_b1, c2_W2, c2_b2,
           o_W1, o_b1, o_W2, o_b2, o_W3, o_b3):
    row = lambda v: v.reshape(1, -1)

    feats1 = pl.pallas_call(
        _conv1_body,
        grid=(B,),
        in_specs=[
            pl.BlockSpec((P, PFC_DIM), lambda b: (b, 0)),
            _full(pe_W1.shape), _full((1, HID)),
            _full(pe_W2.shape), _full((1, HID)),
            _full(c1_W1.shape), _full((1, HID)),
            _full(c1_W2.shape), _full((1, HID)),
        ],
        out_specs=pl.BlockSpec((P, HID), lambda b: (b, 0)),
        out_shape=jax.ShapeDtypeStruct((N, HID), _F32),
    )(x_pfc, pe_W1, row(pe_b1), pe_W2, row(pe_b2),
      c1_W1, row(c1_b1), c1_W2, row(c1_b2))

    concat_feats = jnp.concatenate([x_pfc, feats1], axis=1)
    charged_idx = jnp.nonzero(x_pfc[:, 11] != 0, size=N, fill_value=0)[0]
    charged = concat_feats[charged_idx]

    dcat = PFC_DIM + HID
    out = pl.pallas_call(
        _conv2_body,
        grid=(B,),
        in_specs=[
            pl.BlockSpec((P, dcat), lambda b: (b, 0)),
            pl.BlockSpec((P, dcat), lambda b: (b, 0)),
            _full(c2_W1.shape), _full((1, HID)),
            _full(c2_W2.shape), _full((1, HID)),
            _full(o_W1.shape), _full((1, 32)),
            _full(o_W2.shape), _full((1, 4)),
            _full(o_W3.shape), _full((1, 1)),
        ],
        out_specs=pl.BlockSpec((P, 1), lambda b: (b, 0)),
        out_shape=jax.ShapeDtypeStruct((N, 1), _F32),
    )(concat_feats, charged, c2_W1, row(c2_b1), c2_W2, row(c2_b2),
      o_W1, row(o_b1), o_W2, row(o_b2), o_W3, row(o_b3))

    x_vtx_enc = pl.pallas_call(
        _vtx_body,
        out_shape=jax.ShapeDtypeStruct((NV, HID), _F32),
    )(x_vtx, ve_W1, row(ve_b1), ve_W2, row(ve_b2))

    return (out, batch_pfc, feats1, x_vtx_enc)
